# baseline (speedup 1.0000x reference)
import jax
import jax.numpy as jnp
from jax import lax
from jax.experimental import pallas as pl
from jax.experimental.pallas import tpu as pltpu
from jax.experimental.pallas import tpu_sc as plsc

N = 10000
N2 = 10240
E = 320000
D = 128
R = 8
BN = 512
BNC = 400
CHUNK = 128
NW = 32
NCHUNKS = E // CHUNK
CPW = NCHUNKS // NW
EXTRA = NCHUNKS - CPW * NW
TROWS = N2 // 16
SS = 3
NSS = CPW // SS
DW = 16
DSS = 6
NDSS = CPW // DSS


def _prep_body(x_ref, rw_ref, lw_ref, xw_ref, lm_ref):
    xb = x_ref[...]
    lm_ref[...] = jnp.dot(xb, lw_ref[...], preferred_element_type=jnp.float32)
    for r in range(R):
        xw_ref[:, r, :] = jnp.dot(xb, rw_ref[r],
                                  preferred_element_type=jnp.float32)


def _combine_body(p_ref, dg_ref, lm_ref, o_ref):
    s = p_ref[0] + p_ref[1]
    d = dg_ref[0] + dg_ref[1]
    norm = 1.0 / jnp.maximum(d[:, 0:1], 1.0)
    o_ref[...] = s * norm + lm_ref[...]


def _deg_body(ei_hbm, degout_hbm, dstb1_v, dstb_v, ones_v, zbuf_v, deg_sh,
              ssem):
    cid = lax.axis_index("c")
    sid = lax.axis_index("s")
    wid = cid * 16 + sid

    def fill(j, carry):
        zbuf_v[j, pl.ds(0, DW)] = jnp.zeros((DW,), jnp.float32)
        ones_v[j, pl.ds(0, DW)] = jnp.ones((DW,), jnp.float32)
        return carry
    lax.fori_loop(0, 16, fill, 0)

    def fill2(j, carry):
        ones_v[16 + j, pl.ds(0, DW)] = jnp.ones((DW,), jnp.float32)
        return carry
    lax.fori_loop(0, CHUNK - 16, fill2, 0)

    row0 = sid * TROWS

    def zrow(j, carry):
        pltpu.sync_copy(zbuf_v, deg_sh.at[pl.ds(row0 + j * 16, 16)])
        return carry
    lax.fori_loop(0, TROWS // 16, zrow, 0)

    plsc.subcore_barrier()

    c0 = wid * CPW

    def superstep(u, carry):
        eb = pl.multiple_of((c0 + u * DSS) * CHUNK, CHUNK)
        pltpu.sync_copy(ei_hbm.at[1, pl.ds(eb, DSS * CHUNK)], dstb1_v)
        for k in range(DSS):
            for i in range(CHUNK // 16):
                dstb_v[k, pl.ds(i * 16, 16)] = dstb1_v[
                    pl.ds(k * CHUNK + i * 16, 16)]
        for k in range(DSS):
            pltpu.async_copy(ones_v, deg_sh.at[dstb_v.at[k]], ssem, add=True)
        for k in range(DSS):
            pltpu.make_async_copy(ones_v, deg_sh.at[dstb_v.at[k]],
                                  ssem).wait()
        return carry
    lax.fori_loop(0, NDSS, superstep, 0)

    @pl.when(wid < EXTRA)
    def _extra():
        ex = pl.multiple_of((NW * CPW + wid) * CHUNK, CHUNK)
        pltpu.sync_copy(ei_hbm.at[1, pl.ds(ex, CHUNK)],
                        dstb1_v.at[pl.ds(0, CHUNK)])
        for i in range(CHUNK // 16):
            dstb_v[0, pl.ds(i * 16, 16)] = dstb1_v[pl.ds(i * 16, 16)]
        pltpu.sync_copy(ones_v, deg_sh.at[dstb_v.at[0]], add=True)

    plsc.subcore_barrier()

    pltpu.sync_copy(deg_sh.at[pl.ds(row0, TROWS)],
                    degout_hbm.at[cid, pl.ds(row0, TROWS)])


def _edge_body(ei_hbm, typ_hbm, xw_hbm, out_hbm,
               srcb_v, typb_v, dstb1_v, dstb_v, gidx_v, rows_v, zbuf_v,
               agg_sh, gsemA, ssemA, gsemB, ssemB, fsemA, fsemB):
    cid = lax.axis_index("c")
    sid = lax.axis_index("s")
    wid = cid * 16 + sid

    def zb(j, carry):
        for k in range(D // 16):
            zbuf_v[j, pl.ds(k * 16, 16)] = jnp.zeros((16,), jnp.float32)
        return carry
    lax.fori_loop(0, 16, zb, 0)

    row0 = sid * TROWS

    def zrow(j, carry):
        pltpu.sync_copy(zbuf_v, agg_sh.at[pl.ds(row0 + j * 16, 16)])
        return carry
    lax.fori_loop(0, TROWS // 16, zrow, 0)

    plsc.subcore_barrier()

    c0 = wid * CPW

    def gather(slot, gsem):
        return pltpu.async_copy(xw_hbm.at[gidx_v.at[slot]], rows_v.at[slot],
                                gsem)

    def scatter(slot, ssem):
        return pltpu.async_copy(rows_v.at[slot], agg_sh.at[dstb_v.at[slot]],
                                ssem, add=True)

    def scatter_wait(slot, ssem):
        pltpu.make_async_copy(rows_v.at[slot], agg_sh.at[dstb_v.at[slot]],
                              ssem).wait()

    def gather_wait(slot, gsem):
        pltpu.make_async_copy(xw_hbm.at[gidx_v.at[slot]], rows_v.at[slot],
                              gsem).wait()

    fsem = [fsemA, fsemB]
    gsem = [gsemA, gsemB]
    ssem = [ssemA, ssemB]

    def fetch_refs(u, sp):
        eb = pl.multiple_of((c0 + u) * CHUNK, CHUNK)
        return [
            (ei_hbm.at[0, pl.ds(eb, CHUNK)], srcb_v.at[sp]),
            (typ_hbm.at[pl.ds(eb, CHUNK)], typb_v.at[sp]),
            (ei_hbm.at[1, pl.ds(eb, CHUNK)], dstb1_v.at[sp]),
        ]

    def fetch_start(u, sp):
        for s, d in fetch_refs(u, sp):
            pltpu.async_copy(s, d, fsem[sp])

    def fetch_wait(u, sp):
        for s, d in fetch_refs(u, sp):
            pltpu.make_async_copy(s, d, fsem[sp]).wait()

    def compute_idx(sp):
        for i in range(CHUNK // 16):
            s16 = srcb_v[sp, pl.ds(i * 16, 16)]
            t16 = typb_v[sp, pl.ds(i * 16, 16)]
            gidx_v[sp, pl.ds(i * 16, 16)] = s16 * R + t16
            dstb_v[sp, pl.ds(i * 16, 16)] = dstb1_v[sp, pl.ds(i * 16, 16)]

    def step(c, p, first=False, fire_next=True, fire_fetch2=True):
        q = 1 - p
        if not first:
            scatter_wait(q, ssem[q])
        if fire_next:
            fetch_wait(c + 1, q)
            compute_idx(q)
            if fire_fetch2:
                fetch_start(c + 2, p)
            gather(q, gsem[q])
        gather_wait(p, gsem[p])
        scatter(p, ssem[p])

    fetch_start(0, 0)
    fetch_wait(0, 0)
    compute_idx(0)
    gather(0, gsem[0])
    fetch_start(1, 1)

    step(0, 0, first=True)

    def pair(i, carry):
        step(2 * i + 1, 1)
        step(2 * i + 2, 0)
        return carry
    lax.fori_loop(0, (CPW - 4) // 2, pair, 0)

    step(CPW - 3, 1)
    step(CPW - 2, 0, fire_fetch2=False)
    step(CPW - 1, 1, fire_next=False)
    scatter_wait(1, ssem[1])

    @pl.when(wid < EXTRA)
    def _extra():
        for s, d in fetch_refs(NW * CPW + wid - c0, 0):
            pltpu.sync_copy(s, d)
        compute_idx(0)
        gather(0, gsem[0])
        gather_wait(0, gsem[0])
        scatter(0, ssem[0])
        scatter_wait(0, ssem[0])

    plsc.subcore_barrier()

    pltpu.sync_copy(agg_sh.at[pl.ds(row0, TROWS)],
                    out_hbm.at[cid, pl.ds(row0, TROWS)])


def kernel(x, edge_index, edge_type, rel_weight, loop_weight):

    mesh = plsc.VectorSubcoreMesh(core_axis_name="c", subcore_axis_name="s",
                                  num_cores=2, num_subcores=16)

    deg = pl.kernel(
        _deg_body,
        out_type=jax.ShapeDtypeStruct((2, N2, DW), jnp.float32),
        mesh=mesh,
        scratch_types=[
            pltpu.VMEM((DSS * CHUNK,), jnp.int32),
            pltpu.VMEM((DSS, CHUNK), jnp.int32),
            pltpu.VMEM((CHUNK, DW), jnp.float32),
            pltpu.VMEM((16, DW), jnp.float32),
            pltpu.VMEM_SHARED((N2, DW), jnp.float32),
            pltpu.SemaphoreType.DMA,
        ],
    )
    degp = deg(edge_index)

    xw, lm = pl.pallas_call(
        _prep_body,
        grid=(N2 // BN,),
        in_specs=[
            pl.BlockSpec((BN, D), lambda i: (i, 0)),
            pl.BlockSpec((R, D, D), lambda i: (0, 0, 0)),
            pl.BlockSpec((D, D), lambda i: (0, 0)),
        ],
        out_specs=[
            pl.BlockSpec((BN, R, D), lambda i: (i, 0, 0)),
            pl.BlockSpec((BN, D), lambda i: (i, 0)),
        ],
        out_shape=[
            jax.ShapeDtypeStruct((N2, R, D), jnp.float32),
            jax.ShapeDtypeStruct((N2, D), jnp.float32),
        ],
    )(x, rel_weight, loop_weight)

    xw_flat = xw.reshape(N2 * R, D)

    edge = pl.kernel(
        _edge_body,
        out_type=jax.ShapeDtypeStruct((2, N2, D), jnp.float32),
        mesh=mesh,
        scratch_types=[
            pltpu.VMEM((2, CHUNK), jnp.int32),
            pltpu.VMEM((2, CHUNK), jnp.int32),
            pltpu.VMEM((2, CHUNK), jnp.int32),
            pltpu.VMEM((2, CHUNK), jnp.int32),
            pltpu.VMEM((2, CHUNK), jnp.int32),
            pltpu.VMEM((2, CHUNK, D), jnp.float32),
            pltpu.VMEM((16, D), jnp.float32),
            pltpu.VMEM_SHARED((N2, D), jnp.float32),
            pltpu.SemaphoreType.DMA,
            pltpu.SemaphoreType.DMA,
            pltpu.SemaphoreType.DMA,
            pltpu.SemaphoreType.DMA,
            pltpu.SemaphoreType.DMA,
            pltpu.SemaphoreType.DMA,
        ],
    )
    parts = edge(edge_index, edge_type, xw_flat)

    out = pl.pallas_call(
        _combine_body,
        grid=(N // BNC,),
        in_specs=[
            pl.BlockSpec((2, BNC, D), lambda i: (0, i, 0)),
            pl.BlockSpec((2, BNC, DW), lambda i: (0, i, 0)),
            pl.BlockSpec((BNC, D), lambda i: (i, 0)),
        ],
        out_specs=pl.BlockSpec((BNC, D), lambda i: (i, 0)),
        out_shape=jax.ShapeDtypeStruct((N, D), jnp.float32),
    )(parts, degp, lm)
    return out


# speedup vs baseline: 1.1479x; 1.1479x over previous
import jax
import jax.numpy as jnp
from jax import lax
from jax.experimental import pallas as pl
from jax.experimental.pallas import tpu as pltpu
from jax.experimental.pallas import tpu_sc as plsc

N = 10000
N2 = 10240
E = 320000
D = 128
R = 8
BN = 512
BNC = 400
CHUNK = 128
NW = 32
NCHUNKS = E // CHUNK
CPW = NCHUNKS // NW
EXTRA = NCHUNKS - CPW * NW
TROWS = N2 // 16
SS = 3
NSS = CPW // SS
DW = 16
DSS = 6
NDSS = CPW // DSS


def _prep_body(x_ref, rw_ref, lw_ref, xw_ref, lm_ref):
    xb = x_ref[...]
    lm_ref[...] = jnp.dot(xb, lw_ref[...], preferred_element_type=jnp.float32)
    for r in range(R):
        xw_ref[:, r, :] = jnp.dot(xb, rw_ref[r],
                                  preferred_element_type=jnp.float32)


def _combine_body(p_ref, dg_ref, lm_ref, o_ref):
    s = p_ref[0] + p_ref[1]
    d = dg_ref[0] + dg_ref[1]
    norm = 1.0 / jnp.maximum(d[:, 0:1], 1.0)
    o_ref[...] = s * norm + lm_ref[...]


def _dst_body(ei_ref, o_ref):
    o_ref[...] = ei_ref[1].reshape(NCHUNKS, CHUNK)


def _deg_body(dst_hbm, degout_hbm, dstb_v, ones_v, zbuf_v, deg_sh, ssem):
    cid = lax.axis_index("c")
    sid = lax.axis_index("s")
    wid = cid * 16 + sid

    def fill(j, carry):
        zbuf_v[j, pl.ds(0, DW)] = jnp.zeros((DW,), jnp.float32)
        ones_v[j, pl.ds(0, DW)] = jnp.ones((DW,), jnp.float32)
        return carry
    lax.fori_loop(0, 16, fill, 0)

    def fill2(j, carry):
        ones_v[16 + j, pl.ds(0, DW)] = jnp.ones((DW,), jnp.float32)
        return carry
    lax.fori_loop(0, CHUNK - 16, fill2, 0)

    row0 = sid * TROWS

    def zrow(j, carry):
        pltpu.sync_copy(zbuf_v, deg_sh.at[pl.ds(row0 + j * 16, 16)])
        return carry
    lax.fori_loop(0, TROWS // 16, zrow, 0)

    plsc.subcore_barrier()

    c0 = wid * CPW

    def superstep(u, carry):
        cb = pl.multiple_of(c0 + u * DSS, 2)
        pltpu.sync_copy(dst_hbm.at[pl.ds(cb, DSS)], dstb_v)
        for k in range(DSS):
            pltpu.async_copy(ones_v, deg_sh.at[dstb_v.at[k]], ssem, add=True)
        for k in range(DSS):
            pltpu.make_async_copy(ones_v, deg_sh.at[dstb_v.at[k]],
                                  ssem).wait()
        return carry
    lax.fori_loop(0, NDSS, superstep, 0)

    @pl.when(wid < EXTRA)
    def _extra():
        cx = pl.multiple_of(NW * CPW + wid, 1)
        pltpu.sync_copy(dst_hbm.at[pl.ds(cx, 1)], dstb_v.at[pl.ds(0, 1)])
        pltpu.sync_copy(ones_v, deg_sh.at[dstb_v.at[0]], add=True)

    plsc.subcore_barrier()

    pltpu.sync_copy(deg_sh.at[pl.ds(row0, TROWS)],
                    degout_hbm.at[cid, pl.ds(row0, TROWS)])


def _edge_body(ei_hbm, typ_hbm, xw_hbm, out_hbm,
               srcb_v, typb_v, dstb1_v, dstb_v, gidx_v, rows_v, zbuf_v,
               agg_sh, gsemA, ssemA, gsemB, ssemB, fsemA, fsemB):
    cid = lax.axis_index("c")
    sid = lax.axis_index("s")
    wid = cid * 16 + sid

    def zb(j, carry):
        for k in range(D // 16):
            zbuf_v[j, pl.ds(k * 16, 16)] = jnp.zeros((16,), jnp.float32)
        return carry
    lax.fori_loop(0, 16, zb, 0)

    row0 = sid * TROWS

    def zrow(j, carry):
        pltpu.sync_copy(zbuf_v, agg_sh.at[pl.ds(row0 + j * 16, 16)])
        return carry
    lax.fori_loop(0, TROWS // 16, zrow, 0)

    plsc.subcore_barrier()

    c0 = wid * CPW

    def gather(slot, gsem):
        return pltpu.async_copy(xw_hbm.at[gidx_v.at[slot]], rows_v.at[slot],
                                gsem)

    def scatter(slot, ssem):
        return pltpu.async_copy(rows_v.at[slot], agg_sh.at[dstb_v.at[slot]],
                                ssem, add=True)

    def scatter_wait(slot, ssem):
        pltpu.make_async_copy(rows_v.at[slot], agg_sh.at[dstb_v.at[slot]],
                              ssem).wait()

    def gather_wait(slot, gsem):
        pltpu.make_async_copy(xw_hbm.at[gidx_v.at[slot]], rows_v.at[slot],
                              gsem).wait()

    fsem = [fsemA, fsemB]
    gsem = [gsemA, gsemB]
    ssem = [ssemA, ssemB]

    def fetch_refs(u, sp):
        eb = pl.multiple_of((c0 + u) * CHUNK, CHUNK)
        return [
            (ei_hbm.at[0, pl.ds(eb, CHUNK)], srcb_v.at[sp]),
            (typ_hbm.at[pl.ds(eb, CHUNK)], typb_v.at[sp]),
            (ei_hbm.at[1, pl.ds(eb, CHUNK)], dstb1_v.at[sp]),
        ]

    def fetch_start(u, sp):
        for s, d in fetch_refs(u, sp):
            pltpu.async_copy(s, d, fsem[sp])

    def fetch_wait(u, sp):
        for s, d in fetch_refs(u, sp):
            pltpu.make_async_copy(s, d, fsem[sp]).wait()

    def compute_idx(sp):
        for i in range(CHUNK // 16):
            s16 = srcb_v[sp, pl.ds(i * 16, 16)]
            t16 = typb_v[sp, pl.ds(i * 16, 16)]
            gidx_v[sp, pl.ds(i * 16, 16)] = s16 * R + t16
            dstb_v[sp, pl.ds(i * 16, 16)] = dstb1_v[sp, pl.ds(i * 16, 16)]

    def step(c, p, first=False, fire_next=True, fire_fetch2=True):
        q = 1 - p
        if not first:
            scatter_wait(q, ssem[q])
        if fire_next:
            fetch_wait(c + 1, q)
            compute_idx(q)
            if fire_fetch2:
                fetch_start(c + 2, p)
            gather(q, gsem[q])
        gather_wait(p, gsem[p])
        scatter(p, ssem[p])

    fetch_start(0, 0)
    fetch_wait(0, 0)
    compute_idx(0)
    gather(0, gsem[0])
    fetch_start(1, 1)

    step(0, 0, first=True)

    def pair(i, carry):
        step(2 * i + 1, 1)
        step(2 * i + 2, 0)
        return carry
    lax.fori_loop(0, (CPW - 4) // 2, pair, 0)

    step(CPW - 3, 1)
    step(CPW - 2, 0, fire_fetch2=False)
    step(CPW - 1, 1, fire_next=False)
    scatter_wait(1, ssem[1])

    @pl.when(wid < EXTRA)
    def _extra():
        for s, d in fetch_refs(NW * CPW + wid - c0, 0):
            pltpu.sync_copy(s, d)
        compute_idx(0)
        gather(0, gsem[0])
        gather_wait(0, gsem[0])
        scatter(0, ssem[0])
        scatter_wait(0, ssem[0])

    plsc.subcore_barrier()

    pltpu.sync_copy(agg_sh.at[pl.ds(row0, TROWS)],
                    out_hbm.at[cid, pl.ds(row0, TROWS)])


def kernel(x, edge_index, edge_type, rel_weight, loop_weight):
    dst2 = pl.pallas_call(
        _dst_body,
        out_shape=jax.ShapeDtypeStruct((NCHUNKS, CHUNK), jnp.int32),
    )(edge_index)

    mesh = plsc.VectorSubcoreMesh(core_axis_name="c", subcore_axis_name="s",
                                  num_cores=2, num_subcores=16)

    deg = pl.kernel(
        _deg_body,
        out_type=jax.ShapeDtypeStruct((2, N2, DW), jnp.float32),
        mesh=mesh,
        compiler_params=pltpu.CompilerParams(use_tc_tiling_on_sc=False),
        scratch_types=[
            pltpu.VMEM((DSS, CHUNK), jnp.int32),
            pltpu.VMEM((CHUNK, DW), jnp.float32),
            pltpu.VMEM((16, DW), jnp.float32),
            pltpu.VMEM_SHARED((N2, DW), jnp.float32),
            pltpu.SemaphoreType.DMA,
        ],
    )
    degp = deg(dst2)

    xw, lm = pl.pallas_call(
        _prep_body,
        grid=(N2 // BN,),
        in_specs=[
            pl.BlockSpec((BN, D), lambda i: (i, 0)),
            pl.BlockSpec((R, D, D), lambda i: (0, 0, 0)),
            pl.BlockSpec((D, D), lambda i: (0, 0)),
        ],
        out_specs=[
            pl.BlockSpec((BN, R, D), lambda i: (i, 0, 0)),
            pl.BlockSpec((BN, D), lambda i: (i, 0)),
        ],
        out_shape=[
            jax.ShapeDtypeStruct((N2, R, D), jnp.float32),
            jax.ShapeDtypeStruct((N2, D), jnp.float32),
        ],
    )(x, rel_weight, loop_weight)

    xw_flat = xw.reshape(N2 * R, D)

    edge = pl.kernel(
        _edge_body,
        out_type=jax.ShapeDtypeStruct((2, N2, D), jnp.float32),
        mesh=mesh,
        scratch_types=[
            pltpu.VMEM((2, CHUNK), jnp.int32),
            pltpu.VMEM((2, CHUNK), jnp.int32),
            pltpu.VMEM((2, CHUNK), jnp.int32),
            pltpu.VMEM((2, CHUNK), jnp.int32),
            pltpu.VMEM((2, CHUNK), jnp.int32),
            pltpu.VMEM((2, CHUNK, D), jnp.float32),
            pltpu.VMEM((16, D), jnp.float32),
            pltpu.VMEM_SHARED((N2, D), jnp.float32),
            pltpu.SemaphoreType.DMA,
            pltpu.SemaphoreType.DMA,
            pltpu.SemaphoreType.DMA,
            pltpu.SemaphoreType.DMA,
            pltpu.SemaphoreType.DMA,
            pltpu.SemaphoreType.DMA,
        ],
    )
    parts = edge(edge_index, edge_type, xw_flat)

    out = pl.pallas_call(
        _combine_body,
        grid=(N // BNC,),
        in_specs=[
            pl.BlockSpec((2, BNC, D), lambda i: (0, i, 0)),
            pl.BlockSpec((2, BNC, DW), lambda i: (0, i, 0)),
            pl.BlockSpec((BNC, D), lambda i: (i, 0)),
        ],
        out_specs=pl.BlockSpec((BNC, D), lambda i: (i, 0)),
        out_shape=jax.ShapeDtypeStruct((N, D), jnp.float32),
    )(parts, degp, lm)
    return out


# speedup vs baseline: 1.2062x; 1.0508x over previous
import jax
import jax.numpy as jnp
from jax import lax
from jax.experimental import pallas as pl
from jax.experimental.pallas import tpu as pltpu
from jax.experimental.pallas import tpu_sc as plsc

N = 10000
N2 = 10240
E = 320000
D = 128
R = 8
BN = 512
BNC = 2000
CHUNK = 128
NW = 32
NCHUNKS = E // CHUNK
CPW = NCHUNKS // NW
EXTRA = NCHUNKS - CPW * NW
TROWS = N2 // 16
SS = 3
NSS = CPW // SS
DW = 16
DSS = 6
NDSS = CPW // DSS


def _prep_body(x_ref, rw_ref, lw_ref, xw_ref, lm_ref):
    xb = x_ref[...]
    lm_ref[...] = jnp.dot(xb, lw_ref[...], preferred_element_type=jnp.float32)
    for r in range(R):
        xw_ref[:, r, :] = jnp.dot(xb, rw_ref[r],
                                  preferred_element_type=jnp.float32)


def _combine_body(p_ref, dg_ref, lm_ref, o_ref):
    s = p_ref[0] + p_ref[1]
    d = dg_ref[0] + dg_ref[1]
    norm = 1.0 / jnp.maximum(d[:, 0:1], 1.0)
    o_ref[...] = s * norm + lm_ref[...]


def _dst_body(ei_ref, o_ref):
    o_ref[...] = ei_ref[1].reshape(NCHUNKS, CHUNK)


def _deg_body(dst_hbm, degout_hbm, dstb_v, ones_v, zbuf_v, deg_sh, ssem):
    cid = lax.axis_index("c")
    sid = lax.axis_index("s")
    wid = cid * 16 + sid

    def fill(j, carry):
        zbuf_v[j, pl.ds(0, DW)] = jnp.zeros((DW,), jnp.float32)
        ones_v[j, pl.ds(0, DW)] = jnp.ones((DW,), jnp.float32)
        return carry
    lax.fori_loop(0, 16, fill, 0)

    def fill2(j, carry):
        ones_v[16 + j, pl.ds(0, DW)] = jnp.ones((DW,), jnp.float32)
        return carry
    lax.fori_loop(0, CHUNK - 16, fill2, 0)

    row0 = sid * TROWS

    def zrow(j, carry):
        pltpu.sync_copy(zbuf_v, deg_sh.at[pl.ds(row0 + j * 16, 16)])
        return carry
    lax.fori_loop(0, TROWS // 16, zrow, 0)

    plsc.subcore_barrier()

    c0 = wid * CPW

    def superstep(u, carry):
        cb = pl.multiple_of(c0 + u * DSS, 2)
        pltpu.sync_copy(dst_hbm.at[pl.ds(cb, DSS)], dstb_v)
        for k in range(DSS):
            pltpu.async_copy(ones_v, deg_sh.at[dstb_v.at[k]], ssem, add=True)
        for k in range(DSS):
            pltpu.make_async_copy(ones_v, deg_sh.at[dstb_v.at[k]],
                                  ssem).wait()
        return carry
    lax.fori_loop(0, NDSS, superstep, 0)

    @pl.when(wid < EXTRA)
    def _extra():
        cx = pl.multiple_of(NW * CPW + wid, 1)
        pltpu.sync_copy(dst_hbm.at[pl.ds(cx, 1)], dstb_v.at[pl.ds(0, 1)])
        pltpu.sync_copy(ones_v, deg_sh.at[dstb_v.at[0]], add=True)

    plsc.subcore_barrier()

    pltpu.sync_copy(deg_sh.at[pl.ds(row0, TROWS)],
                    degout_hbm.at[cid, pl.ds(row0, TROWS)])


def _edge_body(ei_hbm, typ_hbm, xw_hbm, out_hbm,
               srcb_v, typb_v, dstb1_v, dstb_v, gidx_v, rows_v, zbuf_v,
               agg_sh, gsemA, ssemA, gsemB, ssemB, fsemA, fsemB):
    cid = lax.axis_index("c")
    sid = lax.axis_index("s")
    wid = cid * 16 + sid

    def zb(j, carry):
        for k in range(D // 16):
            zbuf_v[j, pl.ds(k * 16, 16)] = jnp.zeros((16,), jnp.float32)
        return carry
    lax.fori_loop(0, 16, zb, 0)

    row0 = sid * TROWS

    def zrow(j, carry):
        pltpu.sync_copy(zbuf_v, agg_sh.at[pl.ds(row0 + j * 16, 16)])
        return carry
    lax.fori_loop(0, TROWS // 16, zrow, 0)

    plsc.subcore_barrier()

    c0 = wid * CPW

    def gather(slot, gsem):
        return pltpu.async_copy(xw_hbm.at[gidx_v.at[slot]], rows_v.at[slot],
                                gsem)

    def scatter(slot, ssem):
        return pltpu.async_copy(rows_v.at[slot], agg_sh.at[dstb_v.at[slot]],
                                ssem, add=True)

    def scatter_wait(slot, ssem):
        pltpu.make_async_copy(rows_v.at[slot], agg_sh.at[dstb_v.at[slot]],
                              ssem).wait()

    def gather_wait(slot, gsem):
        pltpu.make_async_copy(xw_hbm.at[gidx_v.at[slot]], rows_v.at[slot],
                              gsem).wait()

    fsem = [fsemA, fsemB]
    gsem = [gsemA, gsemB]
    ssem = [ssemA, ssemB]

    def fetch_refs(u, sp):
        eb = pl.multiple_of((c0 + u) * CHUNK, CHUNK)
        return [
            (ei_hbm.at[0, pl.ds(eb, CHUNK)], srcb_v.at[sp]),
            (typ_hbm.at[pl.ds(eb, CHUNK)], typb_v.at[sp]),
            (ei_hbm.at[1, pl.ds(eb, CHUNK)], dstb1_v.at[sp]),
        ]

    def fetch_start(u, sp):
        for s, d in fetch_refs(u, sp):
            pltpu.async_copy(s, d, fsem[sp])

    def fetch_wait(u, sp):
        for s, d in fetch_refs(u, sp):
            pltpu.make_async_copy(s, d, fsem[sp]).wait()

    def compute_idx(sp):
        for i in range(CHUNK // 16):
            s16 = srcb_v[sp, pl.ds(i * 16, 16)]
            t16 = typb_v[sp, pl.ds(i * 16, 16)]
            gidx_v[sp, pl.ds(i * 16, 16)] = s16 * R + t16
            dstb_v[sp, pl.ds(i * 16, 16)] = dstb1_v[sp, pl.ds(i * 16, 16)]

    def step(c, p, first=False, fire_next=True, fire_fetch2=True):
        q = 1 - p
        if not first:
            scatter_wait(q, ssem[q])
        if fire_next:
            fetch_wait(c + 1, q)
            compute_idx(q)
            if fire_fetch2:
                fetch_start(c + 2, p)
            gather(q, gsem[q])
        gather_wait(p, gsem[p])
        scatter(p, ssem[p])

    fetch_start(0, 0)
    fetch_wait(0, 0)
    compute_idx(0)
    gather(0, gsem[0])
    fetch_start(1, 1)

    step(0, 0, first=True)

    def pair(i, carry):
        step(2 * i + 1, 1)
        step(2 * i + 2, 0)
        return carry
    lax.fori_loop(0, (CPW - 4) // 2, pair, 0)

    step(CPW - 3, 1)
    step(CPW - 2, 0, fire_fetch2=False)
    step(CPW - 1, 1, fire_next=False)
    scatter_wait(1, ssem[1])

    @pl.when(wid < EXTRA)
    def _extra():
        for s, d in fetch_refs(NW * CPW + wid - c0, 0):
            pltpu.sync_copy(s, d)
        compute_idx(0)
        gather(0, gsem[0])
        gather_wait(0, gsem[0])
        scatter(0, ssem[0])
        scatter_wait(0, ssem[0])

    plsc.subcore_barrier()

    pltpu.sync_copy(agg_sh.at[pl.ds(row0, TROWS)],
                    out_hbm.at[cid, pl.ds(row0, TROWS)])


def kernel(x, edge_index, edge_type, rel_weight, loop_weight):
    dst2 = pl.pallas_call(
        _dst_body,
        out_shape=jax.ShapeDtypeStruct((NCHUNKS, CHUNK), jnp.int32),
    )(edge_index)

    mesh = plsc.VectorSubcoreMesh(core_axis_name="c", subcore_axis_name="s",
                                  num_cores=2, num_subcores=16)

    deg = pl.kernel(
        _deg_body,
        out_type=jax.ShapeDtypeStruct((2, N2, DW), jnp.float32),
        mesh=mesh,
        compiler_params=pltpu.CompilerParams(use_tc_tiling_on_sc=False),
        scratch_types=[
            pltpu.VMEM((DSS, CHUNK), jnp.int32),
            pltpu.VMEM((CHUNK, DW), jnp.float32),
            pltpu.VMEM((16, DW), jnp.float32),
            pltpu.VMEM_SHARED((N2, DW), jnp.float32),
            pltpu.SemaphoreType.DMA,
        ],
    )
    degp = deg(dst2)

    xw, lm = pl.pallas_call(
        _prep_body,
        grid=(N2 // BN,),
        in_specs=[
            pl.BlockSpec((BN, D), lambda i: (i, 0)),
            pl.BlockSpec((R, D, D), lambda i: (0, 0, 0)),
            pl.BlockSpec((D, D), lambda i: (0, 0)),
        ],
        out_specs=[
            pl.BlockSpec((BN, R, D), lambda i: (i, 0, 0)),
            pl.BlockSpec((BN, D), lambda i: (i, 0)),
        ],
        out_shape=[
            jax.ShapeDtypeStruct((N2, R, D), jnp.float32),
            jax.ShapeDtypeStruct((N2, D), jnp.float32),
        ],
    )(x, rel_weight, loop_weight)

    xw_flat = xw.reshape(N2 * R, D)

    edge = pl.kernel(
        _edge_body,
        out_type=jax.ShapeDtypeStruct((2, N2, D), jnp.float32),
        mesh=mesh,
        scratch_types=[
            pltpu.VMEM((2, CHUNK), jnp.int32),
            pltpu.VMEM((2, CHUNK), jnp.int32),
            pltpu.VMEM((2, CHUNK), jnp.int32),
            pltpu.VMEM((2, CHUNK), jnp.int32),
            pltpu.VMEM((2, CHUNK), jnp.int32),
            pltpu.VMEM((2, CHUNK, D), jnp.float32),
            pltpu.VMEM((16, D), jnp.float32),
            pltpu.VMEM_SHARED((N2, D), jnp.float32),
            pltpu.SemaphoreType.DMA,
            pltpu.SemaphoreType.DMA,
            pltpu.SemaphoreType.DMA,
            pltpu.SemaphoreType.DMA,
            pltpu.SemaphoreType.DMA,
            pltpu.SemaphoreType.DMA,
        ],
    )
    parts = edge(edge_index, edge_type, xw_flat)

    out = pl.pallas_call(
        _combine_body,
        grid=(N // BNC,),
        in_specs=[
            pl.BlockSpec((2, BNC, D), lambda i: (0, i, 0)),
            pl.BlockSpec((2, BNC, DW), lambda i: (0, i, 0)),
            pl.BlockSpec((BNC, D), lambda i: (i, 0)),
        ],
        out_specs=pl.BlockSpec((BNC, D), lambda i: (i, 0)),
        out_shape=jax.ShapeDtypeStruct((N, D), jnp.float32),
    )(parts, degp, lm)
    return out
